# baseline (speedup 1.0000x reference)
import jax
import jax.numpy as jnp
from jax import lax
from jax.experimental import pallas as pl
from jax.experimental.pallas import tpu as pltpu

_NUM_CLASSES = 80
_B = 16
_LEVELS = ((8.0, 64, 4096, 6), (16.0, 32, 1024, 5), (32.0, 16, 256, 4))
_PTS = 5376
_REG_W = 5.0


def _softplus_bce_terms(l):
    return jnp.sum(jnp.maximum(l, 0.0) + jnp.log1p(jnp.exp(-jnp.abs(l))))


def _loss_kernel(p8_ref, p16_ref, p32_ref, reg_ref, obj_ref, cls_ref, out_ref):
    b = pl.program_id(0)
    total = jnp.float32(0.0)
    off = 0
    for (stride, w, s, lw), pref in zip(_LEVELS, (p8_ref, p16_ref, p32_ref)):
        p = pref[0]
        hw = lax.broadcasted_iota(jnp.int32, (1, s), 1)
        gx = (hw & (w - 1)).astype(jnp.float32)
        gy = (hw >> lw).astype(jnp.float32)
        px = (p[0:1, :] + gx) * stride
        py = (p[1:2, :] + gy) * stride
        pw = jnp.exp(p[2:3, :]) * stride
        ph = jnp.exp(p[3:4, :]) * stride

        rt = reg_ref[0, pl.ds(off, s), :]
        rtT = jnp.transpose(rt)
        tx = rtT[0:1, :]
        ty = rtT[1:2, :]
        tw = rtT[2:3, :]
        th = rtT[3:4, :]

        p_l = px - pw * 0.5
        p_t = py - ph * 0.5
        p_r = px + pw * 0.5
        p_b = py + ph * 0.5
        t_l = tx - tw * 0.5
        t_t = ty - th * 0.5
        t_r = tx + tw * 0.5
        t_b = ty + th * 0.5

        tlx = jnp.maximum(p_l, t_l)
        tly = jnp.maximum(p_t, t_t)
        brx = jnp.minimum(p_r, t_r)
        bry = jnp.minimum(p_b, t_b)
        en = ((tlx < brx) & (tly < bry)).astype(jnp.float32)
        inter = (brx - tlx) * (bry - tly) * en
        union = pw * ph + tw * th - inter
        iou = inter / (union + 1e-16)
        ctlx = jnp.minimum(p_l, t_l)
        ctly = jnp.minimum(p_t, t_t)
        cbrx = jnp.maximum(p_r, t_r)
        cbry = jnp.maximum(p_b, t_b)
        area_c = (cbrx - ctlx) * (cbry - ctly)
        giou = iou - (area_c - union) / jnp.maximum(area_c, 1e-16)
        total += _REG_W * jnp.sum(1.0 - jnp.clip(giou, -1.0, 1.0))

        total += _softplus_bce_terms(p) - _softplus_bce_terms(p[0:4, :])

        obj_t = obj_ref[0, pl.ds(off, s), :]
        cls_t = cls_ref[0, pl.ds(off, s), :]
        mo = lax.dot(p, obj_t, preferred_element_type=jnp.float32)
        total -= mo[4, 0]
        mc = lax.dot(p, cls_t, preferred_element_type=jnp.float32)
        row = lax.broadcasted_iota(jnp.int32, (85, 80), 0)
        col = lax.broadcasted_iota(jnp.int32, (85, 80), 1)
        total -= jnp.sum(jnp.where(row == col + 5, mc, 0.0))

        off += s

    total = total * jnp.float32(1.0 / (_B * _PTS))

    @pl.when(b == 0)
    def _init():
        out_ref[...] = total.reshape(1, 1)

    @pl.when(b != 0)
    def _acc():
        out_ref[...] += total.reshape(1, 1)


def kernel(p8, p16, p32, reg_targets, obj_targets, cls_targets):
    p8r = p8.reshape(_B, 85, 64 * 64)
    p16r = p16.reshape(_B, 85, 32 * 32)
    p32r = p32.reshape(_B, 85, 16 * 16)
    reg = reg_targets.reshape(_B, _PTS, 4)
    obj = obj_targets.reshape(_B, _PTS, 1)
    cls = cls_targets.reshape(_B, _PTS, _NUM_CLASSES)

    out = pl.pallas_call(
        _loss_kernel,
        grid=(_B,),
        in_specs=[
            pl.BlockSpec((1, 85, 64 * 64), lambda b: (b, 0, 0)),
            pl.BlockSpec((1, 85, 32 * 32), lambda b: (b, 0, 0)),
            pl.BlockSpec((1, 85, 16 * 16), lambda b: (b, 0, 0)),
            pl.BlockSpec((1, _PTS, 4), lambda b: (b, 0, 0)),
            pl.BlockSpec((1, _PTS, 1), lambda b: (b, 0, 0)),
            pl.BlockSpec((1, _PTS, _NUM_CLASSES), lambda b: (b, 0, 0)),
        ],
        out_specs=pl.BlockSpec((1, 1), lambda b: (0, 0)),
        out_shape=jax.ShapeDtypeStruct((1, 1), jnp.float32),
        compiler_params=pltpu.CompilerParams(
            dimension_semantics=("arbitrary",),
        ),
    )(p8r, p16r, p32r, reg, obj, cls)
    return out[0, 0]
